# baseline (speedup 1.0000x reference)
import jax
import jax.numpy as jnp
from jax import lax
from jax.experimental import pallas as pl

B, Q, L = 64, 100, 256


def _postproc_kernel(logits_ref, boxes_ref, scale_ref, posmap_ref, out_ref):
    x = logits_ref[0]
    m = jnp.max(x, axis=-1, keepdims=True)
    e = jnp.exp(x - m)
    s = jnp.sum(e, axis=-1, keepdims=True)
    p = e / s
    pos = posmap_ref[0] > 1e-6
    masked = jnp.where(pos, p, 0.0)
    score = jnp.max(masked, axis=-1, keepdims=True)

    bx = boxes_ref[0]
    cx = bx[:, 0:1]
    cy = bx[:, 1:2]
    w = bx[:, 2:3]
    h = bx[:, 3:4]
    xyxy = jnp.concatenate(
        [cx - 0.5 * w, cy - 0.5 * h, cx + 0.5 * w, cy + 0.5 * h], axis=-1
    )
    xyxy = xyxy * scale_ref[0]

    ii = lax.broadcasted_iota(jnp.int32, (Q, Q), 0)
    jj = lax.broadcasted_iota(jnp.int32, (Q, Q), 1)
    diag = ii == jj

    score_row = jnp.max(
        jnp.where(diag, jnp.broadcast_to(score, (Q, Q)), 0.0),
        axis=0,
        keepdims=True,
    )

    beats = (score_row > score) | ((score_row == score) & (jj < ii))
    rank = jnp.sum(beats.astype(jnp.int32), axis=1, keepdims=True)

    rank_row = jnp.max(
        jnp.where(diag, jnp.broadcast_to(rank, (Q, Q)), 0),
        axis=0,
        keepdims=True,
    )

    rr = lax.broadcasted_iota(jnp.int32, (Q, Q), 0)
    take = rank_row == rr

    cols = []
    for c in range(4):
        col = xyxy[:, c : c + 1]
        col_row = jnp.sum(
            jnp.where(diag, jnp.broadcast_to(col, (Q, Q)), 0.0),
            axis=0,
            keepdims=True,
        )
        cols.append(
            jnp.sum(
                jnp.where(take, jnp.broadcast_to(col_row, (Q, Q)), 0.0),
                axis=1,
                keepdims=True,
            )
        )
    out_ref[0] = jnp.concatenate(cols, axis=-1)


def kernel(pred_logits, pred_boxes, target_sizes, positive_map, items_per_batch_element):
    del items_per_batch_element
    img_h = target_sizes[:, 0].astype(jnp.float32)
    img_w = target_sizes[:, 1].astype(jnp.float32)
    scale = jnp.stack([img_w, img_h, img_w, img_h], axis=1)[:, None, :]
    posmap = positive_map[:, None, :]

    return pl.pallas_call(
        _postproc_kernel,
        grid=(B,),
        in_specs=[
            pl.BlockSpec((1, Q, L), lambda b: (b, 0, 0)),
            pl.BlockSpec((1, Q, 4), lambda b: (b, 0, 0)),
            pl.BlockSpec((1, 1, 4), lambda b: (b, 0, 0)),
            pl.BlockSpec((1, 1, L), lambda b: (b, 0, 0)),
        ],
        out_specs=pl.BlockSpec((1, Q, 4), lambda b: (b, 0, 0)),
        out_shape=jax.ShapeDtypeStruct((B, Q, 4), jnp.float32),
    )(pred_logits, pred_boxes, scale, posmap)


# speedup vs baseline: 1.7680x; 1.7680x over previous
import jax
import jax.numpy as jnp
from jax import lax
from jax.experimental import pallas as pl

B, Q, L = 64, 100, 256
BB = 8


def _postproc_kernel(logits_ref, boxes_ref, scale_ref, posmap_ref, out_ref):
    x = logits_ref[...]
    m = jnp.max(x, axis=-1, keepdims=True)
    e = jnp.exp(x - m)
    s = jnp.sum(e, axis=-1, keepdims=True)
    pos = posmap_ref[...] > 1e-6
    emax = jnp.max(jnp.where(pos, e, 0.0), axis=-1, keepdims=True)
    score = emax / s

    bx = boxes_ref[...]
    cx = bx[:, :, 0:1]
    cy = bx[:, :, 1:2]
    w = bx[:, :, 2:3]
    h = bx[:, :, 3:4]
    xyxy = jnp.concatenate(
        [cx - 0.5 * w, cy - 0.5 * h, cx + 0.5 * w, cy + 0.5 * h], axis=-1
    )
    xyxy = xyxy * scale_ref[...]

    score_row = jnp.swapaxes(score, 1, 2)
    ii = lax.broadcasted_iota(jnp.int32, (1, Q, Q), 1)
    jj = lax.broadcasted_iota(jnp.int32, (1, Q, Q), 2)

    beats = (score_row > score) | ((score_row == score) & (jj < ii))
    rank = jnp.sum(beats.astype(jnp.int32), axis=2, keepdims=True)

    rr = lax.broadcasted_iota(jnp.int32, (1, 1, Q), 2)
    takeT = (rank == rr).astype(jnp.float32)

    out_ref[...] = lax.dot_general(
        takeT,
        xyxy,
        dimension_numbers=(((1,), (1,)), ((0,), (0,))),
        preferred_element_type=jnp.float32,
        precision=lax.Precision.HIGHEST,
    )


def kernel(pred_logits, pred_boxes, target_sizes, positive_map, items_per_batch_element):
    del items_per_batch_element
    img_h = target_sizes[:, 0].astype(jnp.float32)
    img_w = target_sizes[:, 1].astype(jnp.float32)
    scale = jnp.stack([img_w, img_h, img_w, img_h], axis=1)[:, None, :]
    posmap = positive_map[:, None, :]

    return pl.pallas_call(
        _postproc_kernel,
        grid=(B // BB,),
        in_specs=[
            pl.BlockSpec((BB, Q, L), lambda b: (b, 0, 0)),
            pl.BlockSpec((BB, Q, 4), lambda b: (b, 0, 0)),
            pl.BlockSpec((BB, 1, 4), lambda b: (b, 0, 0)),
            pl.BlockSpec((BB, 1, L), lambda b: (b, 0, 0)),
        ],
        out_specs=pl.BlockSpec((BB, Q, 4), lambda b: (b, 0, 0)),
        out_shape=jax.ShapeDtypeStruct((B, Q, 4), jnp.float32),
    )(pred_logits, pred_boxes, scale, posmap)


# speedup vs baseline: 1.8452x; 1.0436x over previous
import jax
import jax.numpy as jnp
from jax import lax
from jax.experimental import pallas as pl
from jax.experimental.pallas import tpu as pltpu

B, Q, L = 64, 100, 256
BB = 8


def _postproc_kernel(logits_ref, boxes_ref, ts_ref, posmap_ref, out_ref):
    x = logits_ref[...]
    m = jnp.max(x, axis=-1, keepdims=True)
    e = jnp.exp(x - m)
    s = jnp.sum(e, axis=-1, keepdims=True)
    pos = posmap_ref[...][:, None, :] > 1e-6
    emax = jnp.max(jnp.where(pos, e, 0.0), axis=-1, keepdims=True)
    score = emax / s

    ts = ts_ref[...].astype(jnp.float32)
    img_h = ts[:, 0:1][:, None, :]
    img_w = ts[:, 1:2][:, None, :]

    bx = boxes_ref[...]
    cx = bx[:, :, 0:1]
    cy = bx[:, :, 1:2]
    w = bx[:, :, 2:3]
    h = bx[:, :, 3:4]
    xyxy = jnp.concatenate(
        [
            (cx - 0.5 * w) * img_w,
            (cy - 0.5 * h) * img_h,
            (cx + 0.5 * w) * img_w,
            (cy + 0.5 * h) * img_h,
        ],
        axis=-1,
    )

    score_row = jnp.swapaxes(score, 1, 2)
    ii = lax.broadcasted_iota(jnp.int32, (1, Q, Q), 1)
    jj = lax.broadcasted_iota(jnp.int32, (1, Q, Q), 2)

    beats = (score_row > score) | ((score_row == score) & (jj < ii))
    rank = jnp.sum(beats.astype(jnp.int32), axis=2, keepdims=True)

    rr = lax.broadcasted_iota(jnp.int32, (1, 1, Q), 2)
    takeT = (rank == rr).astype(jnp.float32)

    out_ref[...] = lax.dot_general(
        takeT,
        xyxy,
        dimension_numbers=(((1,), (1,)), ((0,), (0,))),
        preferred_element_type=jnp.float32,
        precision=lax.Precision.HIGHEST,
    )


def kernel(pred_logits, pred_boxes, target_sizes, positive_map, items_per_batch_element):
    del items_per_batch_element
    return pl.pallas_call(
        _postproc_kernel,
        grid=(B // BB,),
        in_specs=[
            pl.BlockSpec((BB, Q, L), lambda b: (b, 0, 0)),
            pl.BlockSpec((BB, Q, 4), lambda b: (b, 0, 0)),
            pl.BlockSpec((BB, 2), lambda b: (b, 0)),
            pl.BlockSpec((BB, L), lambda b: (b, 0)),
        ],
        out_specs=pl.BlockSpec((BB, Q, 4), lambda b: (b, 0, 0)),
        out_shape=jax.ShapeDtypeStruct((B, Q, 4), jnp.float32),
        compiler_params=pltpu.CompilerParams(
            dimension_semantics=("parallel",),
        ),
    )(pred_logits, pred_boxes, target_sizes, positive_map)


# speedup vs baseline: 2.0161x; 1.0926x over previous
import jax
import jax.numpy as jnp
from jax import lax
from jax.experimental import pallas as pl
from jax.experimental.pallas import tpu as pltpu

B, Q, L = 64, 100, 256
BB = 16


def _postproc_kernel(logits_ref, boxes_ref, ts_ref, posmap_ref, out_ref):
    x = logits_ref[...]
    m = jnp.max(x, axis=-1, keepdims=True)
    e = jnp.exp(x - m)
    s = jnp.sum(e, axis=-1, keepdims=True)
    pos = posmap_ref[...][:, None, :] > 1e-6
    emax = jnp.max(jnp.where(pos, e, 0.0), axis=-1, keepdims=True)
    score = emax / s

    ts = ts_ref[...].astype(jnp.float32)
    img_h = ts[:, 0:1][:, None, :]
    img_w = ts[:, 1:2][:, None, :]

    bx = boxes_ref[...]
    cx = bx[:, :, 0:1]
    cy = bx[:, :, 1:2]
    w = bx[:, :, 2:3]
    h = bx[:, :, 3:4]
    xyxy = jnp.concatenate(
        [
            (cx - 0.5 * w) * img_w,
            (cy - 0.5 * h) * img_h,
            (cx + 0.5 * w) * img_w,
            (cy + 0.5 * h) * img_h,
        ],
        axis=-1,
    )

    score_row = jnp.swapaxes(score, 1, 2)
    ii = lax.broadcasted_iota(jnp.int32, (1, Q, Q), 1)
    jj = lax.broadcasted_iota(jnp.int32, (1, Q, Q), 2)

    beats = (score_row > score) | ((score_row == score) & (jj < ii))
    rank = jnp.sum(beats.astype(jnp.int32), axis=2, keepdims=True)

    rr = lax.broadcasted_iota(jnp.int32, (1, 1, Q), 2)
    takeT = (rank == rr).astype(jnp.float32)

    out_ref[...] = lax.dot_general(
        takeT,
        xyxy,
        dimension_numbers=(((1,), (1,)), ((0,), (0,))),
        preferred_element_type=jnp.float32,
        precision=lax.Precision.HIGHEST,
    )


def kernel(pred_logits, pred_boxes, target_sizes, positive_map, items_per_batch_element):
    del items_per_batch_element
    return pl.pallas_call(
        _postproc_kernel,
        grid=(B // BB,),
        in_specs=[
            pl.BlockSpec((BB, Q, L), lambda b: (b, 0, 0)),
            pl.BlockSpec((BB, Q, 4), lambda b: (b, 0, 0)),
            pl.BlockSpec((BB, 2), lambda b: (b, 0)),
            pl.BlockSpec((BB, L), lambda b: (b, 0)),
        ],
        out_specs=pl.BlockSpec((BB, Q, 4), lambda b: (b, 0, 0)),
        out_shape=jax.ShapeDtypeStruct((B, Q, 4), jnp.float32),
        compiler_params=pltpu.CompilerParams(
            dimension_semantics=("parallel",),
        ),
    )(pred_logits, pred_boxes, target_sizes, positive_map)


# speedup vs baseline: 2.0698x; 1.0266x over previous
import jax
import jax.numpy as jnp
from jax import lax
from jax.experimental import pallas as pl
from jax.experimental.pallas import tpu as pltpu

B, Q, L = 64, 100, 256
BB = 32


def _postproc_kernel(logits_ref, boxes_ref, ts_ref, posmap_ref, out_ref):
    x = logits_ref[...]
    m = jnp.max(x, axis=-1, keepdims=True)
    e = jnp.exp(x - m)
    s = jnp.sum(e, axis=-1, keepdims=True)
    pos = posmap_ref[...][:, None, :] > 1e-6
    emax = jnp.max(jnp.where(pos, e, 0.0), axis=-1, keepdims=True)
    score = emax / s

    ts = ts_ref[...].astype(jnp.float32)
    img_h = ts[:, 0:1][:, None, :]
    img_w = ts[:, 1:2][:, None, :]

    bx = boxes_ref[...]
    cx = bx[:, :, 0:1]
    cy = bx[:, :, 1:2]
    w = bx[:, :, 2:3]
    h = bx[:, :, 3:4]
    xyxy = jnp.concatenate(
        [
            (cx - 0.5 * w) * img_w,
            (cy - 0.5 * h) * img_h,
            (cx + 0.5 * w) * img_w,
            (cy + 0.5 * h) * img_h,
        ],
        axis=-1,
    )

    score_row = jnp.swapaxes(score, 1, 2)
    ii = lax.broadcasted_iota(jnp.int32, (1, Q, Q), 1)
    jj = lax.broadcasted_iota(jnp.int32, (1, Q, Q), 2)

    beats = (score_row > score) | ((score_row == score) & (jj < ii))
    rank = jnp.sum(beats.astype(jnp.int32), axis=2, keepdims=True)

    rr = lax.broadcasted_iota(jnp.int32, (1, 1, Q), 2)
    takeT = (rank == rr).astype(jnp.float32)

    out_ref[...] = lax.dot_general(
        takeT,
        xyxy,
        dimension_numbers=(((1,), (1,)), ((0,), (0,))),
        preferred_element_type=jnp.float32,
        precision=lax.Precision.HIGHEST,
    )


def kernel(pred_logits, pred_boxes, target_sizes, positive_map, items_per_batch_element):
    del items_per_batch_element
    return pl.pallas_call(
        _postproc_kernel,
        grid=(B // BB,),
        in_specs=[
            pl.BlockSpec((BB, Q, L), lambda b: (b, 0, 0)),
            pl.BlockSpec((BB, Q, 4), lambda b: (b, 0, 0)),
            pl.BlockSpec((BB, 2), lambda b: (b, 0)),
            pl.BlockSpec((BB, L), lambda b: (b, 0)),
        ],
        out_specs=pl.BlockSpec((BB, Q, 4), lambda b: (b, 0, 0)),
        out_shape=jax.ShapeDtypeStruct((B, Q, 4), jnp.float32),
        compiler_params=pltpu.CompilerParams(
            dimension_semantics=("parallel",),
        ),
    )(pred_logits, pred_boxes, target_sizes, positive_map)
